# Initial kernel scaffold; baseline (speedup 1.0000x reference)
import jax
import jax.numpy as jnp
from jax.experimental import pallas as pl


def kernel(x, edge_index, w1a, b1a, w2a, b2a, eps1, w1b, b1b, w2b, b2b, eps2, wfc, bfc):
    raise NotImplementedError("write your pallas kernel here")


# speedup vs baseline: 6.7300x; 6.7300x over previous
import functools

import jax
import jax.numpy as jnp
from jax import lax
from jax.experimental import pallas as pl
from jax.experimental.pallas import tpu as pltpu
from jax.experimental.pallas import tpu_sc as plsc

_N = 10000
_E = 320000
_NS = 16
_CHUNK = 128
_NPAD = 10240
_ZROWS = 128
_RPT = _NPAD // _NS
_NROWS = 2560
_NCH = _NROWS // _NS
_SEG = 32
_W = 64


def _sc_aggregate(nparts: int):
  nphases = nparts // 2
  mesh = plsc.VectorSubcoreMesh(core_axis_name="c", subcore_axis_name="s")

  @functools.partial(
      pl.kernel,
      out_type=jax.ShapeDtypeStruct((nparts, _NPAD, _W), jnp.float32),
      mesh=mesh,
      compiler_params=pltpu.CompilerParams(use_tc_tiling_on_sc=False),
      scratch_types=[
          pltpu.VMEM((_SEG, _CHUNK), jnp.int32),
          pltpu.VMEM((_SEG, _CHUNK), jnp.int32),
          pltpu.VMEM((_CHUNK, _W), jnp.float32),
          pltpu.VMEM((_CHUNK, _W), jnp.float32),
          pltpu.VMEM_SHARED((_NPAD, _W), jnp.float32),
          pltpu.VMEM_SHARED((_NPAD, _W), jnp.float32),
          pltpu.SemaphoreType.DMA,
          pltpu.SemaphoreType.DMA,
          pltpu.SemaphoreType.DMA,
          pltpu.SemaphoreType.DMA,
      ],
  )
  def agg_kernel(*refs):
    qs = refs[:nparts]
    src2, dst2, out = refs[nparts:nparts + 3]
    (src_v, dst_v, rows_a, rows_b, xsp, acc,
     sem_ga, sem_gb, sem_sa, sem_sb) = refs[nparts + 3:]
    c = lax.axis_index("c")
    s = lax.axis_index("s")
    r0 = s * _RPT
    z16 = jnp.zeros((16,), jnp.float32)

    def zrow(r, carry):
      for j in range(_W // 16):
        rows_b[r, pl.ds(j * 16, 16)] = z16
      return carry

    def start_gather(i, rows, sem):
      pltpu.async_copy(xsp.at[src_v.at[i]], rows, sem)

    def wait_gather(rows, sem):
      pltpu.make_async_copy(xsp.at[pl.ds(0, _CHUNK)], rows, sem).wait()

    def start_scatter(i, rows, sem):
      pltpu.async_copy(rows, acc.at[dst_v.at[i]], sem, add=True)

    def wait_scatter(rows, sem):
      pltpu.make_async_copy(rows, acc.at[pl.ds(0, _CHUNK)], sem).wait()

    for phase in range(nphases):
      lax.fori_loop(0, _ZROWS, zrow, 0)
      for jj in range(_RPT // _ZROWS):
        rr = r0 + jj * _ZROWS

        @pl.when(c == 0)
        def _():
          pltpu.sync_copy(qs[2 * phase].at[pl.ds(rr, _ZROWS)], rows_a)

        @pl.when(c == 1)
        def _():
          pltpu.sync_copy(qs[2 * phase + 1].at[pl.ds(rr, _ZROWS)], rows_a)

        pltpu.sync_copy(rows_a, xsp.at[pl.ds(rr, _ZROWS)])
        pltpu.sync_copy(rows_b, acc.at[pl.ds(rr, _ZROWS)])
      plsc.subcore_barrier()

      base = s * _NCH

      def seg_body(g, carry):
        @pl.when(g > 0)
        def _():
          wait_scatter(rows_b, sem_sb)

        row0 = base + g * _SEG
        pltpu.sync_copy(src2.at[pl.ds(row0, _SEG)], src_v)
        pltpu.sync_copy(dst2.at[pl.ds(row0, _SEG)], dst_v)
        start_gather(0, rows_a, sem_ga)

        def pair(p, carry2):
          ia = 2 * p
          ib = 2 * p + 1
          wait_gather(rows_a, sem_ga)
          start_scatter(ia, rows_a, sem_sa)

          @pl.when(p > 0)
          def _():
            wait_scatter(rows_b, sem_sb)

          start_gather(ib, rows_b, sem_gb)
          wait_gather(rows_b, sem_gb)
          start_scatter(ib, rows_b, sem_sb)
          wait_scatter(rows_a, sem_sa)

          @pl.when(ib + 1 < _SEG)
          def _():
            start_gather(ib + 1, rows_a, sem_ga)

          return carry2

        lax.fori_loop(0, _SEG // 2, pair, 0)
        return carry

      lax.fori_loop(0, _NCH // _SEG, seg_body, 0)
      wait_scatter(rows_b, sem_sb)
      plsc.subcore_barrier()

      for jj in range(_RPT // _ZROWS):
        rr = r0 + jj * _ZROWS
        pltpu.sync_copy(acc.at[pl.ds(rr, _ZROWS)], rows_a)

        @pl.when(c == 0)
        def _():
          pltpu.sync_copy(rows_a, out.at[2 * phase, pl.ds(rr, _ZROWS)])

        @pl.when(c == 1)
        def _():
          pltpu.sync_copy(rows_a, out.at[2 * phase + 1, pl.ds(rr, _ZROWS)])

      if phase + 1 < nphases:
        plsc.subcore_barrier()

  return agg_kernel


def _pad_edges(idx, fill):
  pad = _NROWS * _CHUNK - _E
  return jnp.concatenate([idx, jnp.full((pad,), fill, jnp.int32)]).reshape(
      _NROWS, _CHUNK)


_BLK = 1000


def _mlp1_body(eps_ref, x_ref, agg_ref, w1_ref, b1_ref, w2_ref, b2_ref,
               hl_ref, hr_ref):
  g = (x_ref[...] * (1.0 + eps_ref[0, 0])
       + jnp.concatenate([agg_ref[0], agg_ref[1]], axis=1))
  t = jnp.maximum(
      jnp.dot(g, w1_ref[...], preferred_element_type=jnp.float32)
      + b1_ref[...], 0.0)
  h = jnp.maximum(
      jnp.dot(t, w2_ref[...], preferred_element_type=jnp.float32)
      + b2_ref[...], 0.0)
  hl_ref[...] = h[:, :128]
  hr_ref[...] = h[:, 128:]


def _tc_mlp1(x, agg, w1, b1, w2, b2, eps):
  return pl.pallas_call(
      _mlp1_body,
      grid=(_N // _BLK,),
      in_specs=[
          pl.BlockSpec(memory_space=pltpu.SMEM),
          pl.BlockSpec((_BLK, 128), lambda i: (i, 0)),
          pl.BlockSpec((2, _BLK, _W), lambda i: (0, i, 0)),
          pl.BlockSpec((128, 256), lambda i: (0, 0)),
          pl.BlockSpec((1, 256), lambda i: (0, 0)),
          pl.BlockSpec((256, 256), lambda i: (0, 0)),
          pl.BlockSpec((1, 256), lambda i: (0, 0)),
      ],
      out_specs=[
          pl.BlockSpec((_BLK, 128), lambda i: (i, 0)),
          pl.BlockSpec((_BLK, 128), lambda i: (i, 0)),
      ],
      out_shape=[
          jax.ShapeDtypeStruct((_NPAD, 128), jnp.float32),
          jax.ShapeDtypeStruct((_NPAD, 128), jnp.float32),
      ],
  )(eps, x, agg, w1, b1, w2, b2)


def _mlp2_body(eps_ref, hl_ref, hr_ref, agg_ref, w1_ref, b1_ref, w2_ref,
               b2_ref, wfc_ref, bfc_ref, out_ref, emb_ref):
  e = 1.0 + eps_ref[0, 0]
  g = jnp.concatenate(
      [hl_ref[...] * e + jnp.concatenate([agg_ref[0], agg_ref[1]], axis=1),
       hr_ref[...] * e + jnp.concatenate([agg_ref[2], agg_ref[3]], axis=1)],
      axis=1)
  t = jnp.maximum(
      jnp.dot(g, w1_ref[...], preferred_element_type=jnp.float32)
      + b1_ref[...], 0.0)
  h2 = jnp.maximum(
      jnp.dot(t, w2_ref[...], preferred_element_type=jnp.float32)
      + b2_ref[...], 0.0)
  emb_ref[...] = h2
  out_ref[...] = (
      jnp.dot(h2, wfc_ref[...], preferred_element_type=jnp.float32)
      + bfc_ref[...])


def _tc_mlp2(hl, hr, agg, w1, b1, w2, b2, eps, wfc, bfc):
  return pl.pallas_call(
      _mlp2_body,
      grid=(_N // _BLK,),
      in_specs=[
          pl.BlockSpec(memory_space=pltpu.SMEM),
          pl.BlockSpec((_BLK, 128), lambda i: (i, 0)),
          pl.BlockSpec((_BLK, 128), lambda i: (i, 0)),
          pl.BlockSpec((4, _BLK, _W), lambda i: (0, i, 0)),
          pl.BlockSpec((256, 64), lambda i: (0, 0)),
          pl.BlockSpec((1, 64), lambda i: (0, 0)),
          pl.BlockSpec((64, 64), lambda i: (0, 0)),
          pl.BlockSpec((1, 64), lambda i: (0, 0)),
          pl.BlockSpec((64, 64), lambda i: (0, 0)),
          pl.BlockSpec((1, 64), lambda i: (0, 0)),
      ],
      out_specs=[
          pl.BlockSpec((_BLK, 64), lambda i: (i, 0)),
          pl.BlockSpec((_BLK, 64), lambda i: (i, 0)),
      ],
      out_shape=[
          jax.ShapeDtypeStruct((_N, 64), jnp.float32),
          jax.ShapeDtypeStruct((_N, 64), jnp.float32),
      ],
  )(eps, hl, hr, agg, w1, b1, w2, b2, wfc, bfc)


def kernel(x, edge_index, w1a, b1a, w2a, b2a, eps1, w1b, b1b, w2b, b2b, eps2,
           wfc, bfc):
  ei = edge_index.astype(jnp.int32)
  src2 = _pad_edges(ei[0], 0)
  dst2 = _pad_edges(ei[1], _N)

  xpad = jnp.pad(x, ((0, _NPAD - _N), (0, 0)))
  agg1 = _sc_aggregate(2)(xpad[:, :_W], xpad[:, _W:], src2, dst2)
  hl, hr = _tc_mlp1(x, agg1, w1a, b1a.reshape(1, -1), w2a,
                    b2a.reshape(1, -1), jnp.reshape(eps1, (1, 1)))
  agg2 = _sc_aggregate(4)(hl[:, :_W], hl[:, _W:], hr[:, :_W], hr[:, _W:],
                          src2, dst2)
  out, emb = _tc_mlp2(hl, hr, agg2, w1b, b1b.reshape(1, -1), w2b,
                      b2b.reshape(1, -1), jnp.reshape(eps2, (1, 1)), wfc,
                      bfc.reshape(1, -1))
  return (out, emb)


# speedup vs baseline: 6.8107x; 1.0120x over previous
import functools

import jax
import jax.numpy as jnp
from jax import lax
from jax.experimental import pallas as pl
from jax.experimental.pallas import tpu as pltpu
from jax.experimental.pallas import tpu_sc as plsc

_N = 10000
_E = 320000
_NS = 16
_CHUNK = 128
_NPAD = 10240
_ZROWS = 128
_RPT = _NPAD // _NS
_NROWS = 2560
_NCH = _NROWS // _NS
_SEG = 32
_W = 64


def _sc_aggregate(nparts: int):
  nphases = nparts // 2
  mesh = plsc.VectorSubcoreMesh(core_axis_name="c", subcore_axis_name="s")

  @functools.partial(
      pl.kernel,
      out_type=jax.ShapeDtypeStruct((nparts, _NPAD, _W), jnp.float32),
      mesh=mesh,
      compiler_params=pltpu.CompilerParams(use_tc_tiling_on_sc=False),
      scratch_types=[
          pltpu.VMEM((_SEG, _CHUNK), jnp.int32),
          pltpu.VMEM((_SEG, _CHUNK), jnp.int32),
          pltpu.VMEM((_CHUNK, _W), jnp.float32),
          pltpu.VMEM((_CHUNK, _W), jnp.float32),
          pltpu.VMEM_SHARED((_NPAD, _W), jnp.float32),
          pltpu.VMEM_SHARED((_NPAD, _W), jnp.float32),
          pltpu.SemaphoreType.DMA,
          pltpu.SemaphoreType.DMA,
          pltpu.SemaphoreType.DMA,
          pltpu.SemaphoreType.DMA,
      ],
  )
  def agg_kernel(*refs):
    qs = refs[:nparts]
    src2, dst2, out = refs[nparts:nparts + 3]
    (src_v, dst_v, rows_a, rows_b, xsp, acc,
     sem_ga, sem_gb, sem_sa, sem_sb) = refs[nparts + 3:]
    c = lax.axis_index("c")
    s = lax.axis_index("s")
    r0 = s * _RPT
    z16 = jnp.zeros((16,), jnp.float32)

    def zrow(r, carry):
      for j in range(_W // 16):
        rows_b[r, pl.ds(j * 16, 16)] = z16
      return carry

    def start_gather(i, rows, sem):
      pltpu.async_copy(xsp.at[src_v.at[i]], rows, sem)

    def wait_gather(rows, sem):
      pltpu.make_async_copy(xsp.at[pl.ds(0, _CHUNK)], rows, sem).wait()

    def start_scatter(i, rows, sem):
      pltpu.async_copy(rows, acc.at[dst_v.at[i]], sem, add=True)

    def wait_scatter(rows, sem):
      pltpu.make_async_copy(rows, acc.at[pl.ds(0, _CHUNK)], sem).wait()

    for phase in range(nphases):
      lax.fori_loop(0, _ZROWS, zrow, 0)
      for jj in range(_RPT // _ZROWS):
        rr = r0 + jj * _ZROWS

        @pl.when(c == 0)
        def _():
          pltpu.sync_copy(qs[2 * phase].at[pl.ds(rr, _ZROWS)], rows_a)

        @pl.when(c == 1)
        def _():
          pltpu.sync_copy(qs[2 * phase + 1].at[pl.ds(rr, _ZROWS)], rows_a)

        pltpu.sync_copy(rows_a, xsp.at[pl.ds(rr, _ZROWS)])
        pltpu.sync_copy(rows_b, acc.at[pl.ds(rr, _ZROWS)])
      plsc.subcore_barrier()

      base = s * _NCH

      def seg_body(g, carry):
        @pl.when(g > 0)
        def _():
          wait_scatter(rows_b, sem_sb)

        row0 = base + g * _SEG
        pltpu.sync_copy(src2.at[pl.ds(row0, _SEG)], src_v)
        pltpu.sync_copy(dst2.at[pl.ds(row0, _SEG)], dst_v)
        start_gather(0, rows_a, sem_ga)

        def pair(p, carry2):
          ia = 2 * p
          ib = 2 * p + 1
          wait_gather(rows_a, sem_ga)
          start_scatter(ia, rows_a, sem_sa)

          @pl.when(p > 0)
          def _():
            wait_scatter(rows_b, sem_sb)

          start_gather(ib, rows_b, sem_gb)
          wait_gather(rows_b, sem_gb)
          start_scatter(ib, rows_b, sem_sb)
          wait_scatter(rows_a, sem_sa)

          @pl.when(ib + 1 < _SEG)
          def _():
            start_gather(ib + 1, rows_a, sem_ga)

          return carry2

        lax.fori_loop(0, _SEG // 2, pair, 0)
        return carry

      lax.fori_loop(0, _NCH // _SEG, seg_body, 0)
      wait_scatter(rows_b, sem_sb)
      plsc.subcore_barrier()

      for jj in range(_RPT // _ZROWS):
        rr = r0 + jj * _ZROWS
        pltpu.sync_copy(acc.at[pl.ds(rr, _ZROWS)], rows_a)

        @pl.when(c == 0)
        def _():
          pltpu.sync_copy(rows_a, out.at[2 * phase, pl.ds(rr, _ZROWS)])

        @pl.when(c == 1)
        def _():
          pltpu.sync_copy(rows_a, out.at[2 * phase + 1, pl.ds(rr, _ZROWS)])

      if phase + 1 < nphases:
        plsc.subcore_barrier()

  return agg_kernel


def _pad_edges(idx, fill):
  pad = _NROWS * _CHUNK - _E
  return jnp.concatenate([idx, jnp.full((pad,), fill, jnp.int32)]).reshape(
      _NROWS, _CHUNK)


_BLK = 1000


def _mlp1_body(eps_ref, x_ref, agg_ref, w1_ref, b1_ref, w2_ref, b2_ref,
               q0_ref, q1_ref, q2_ref, q3_ref):
  g = (x_ref[...] * (1.0 + eps_ref[0, 0])
       + jnp.concatenate([agg_ref[0], agg_ref[1]], axis=1))
  t = jnp.maximum(
      jnp.dot(g, w1_ref[...], preferred_element_type=jnp.float32)
      + b1_ref[...], 0.0)
  h = jnp.maximum(
      jnp.dot(t, w2_ref[...], preferred_element_type=jnp.float32)
      + b2_ref[...], 0.0)
  q0_ref[...] = h[:, 0:64]
  q1_ref[...] = h[:, 64:128]
  q2_ref[...] = h[:, 128:192]
  q3_ref[...] = h[:, 192:256]


def _tc_mlp1(x, agg, w1, b1, w2, b2, eps):
  return pl.pallas_call(
      _mlp1_body,
      grid=(_N // _BLK,),
      in_specs=[
          pl.BlockSpec(memory_space=pltpu.SMEM),
          pl.BlockSpec((_BLK, 128), lambda i: (i, 0)),
          pl.BlockSpec((2, _BLK, _W), lambda i: (0, i, 0)),
          pl.BlockSpec((128, 256), lambda i: (0, 0)),
          pl.BlockSpec((1, 256), lambda i: (0, 0)),
          pl.BlockSpec((256, 256), lambda i: (0, 0)),
          pl.BlockSpec((1, 256), lambda i: (0, 0)),
      ],
      out_specs=[
          pl.BlockSpec((_BLK, _W), lambda i: (i, 0)),
          pl.BlockSpec((_BLK, _W), lambda i: (i, 0)),
          pl.BlockSpec((_BLK, _W), lambda i: (i, 0)),
          pl.BlockSpec((_BLK, _W), lambda i: (i, 0)),
      ],
      out_shape=[
          jax.ShapeDtypeStruct((_NPAD, _W), jnp.float32),
          jax.ShapeDtypeStruct((_NPAD, _W), jnp.float32),
          jax.ShapeDtypeStruct((_NPAD, _W), jnp.float32),
          jax.ShapeDtypeStruct((_NPAD, _W), jnp.float32),
      ],
  )(eps, x, agg, w1, b1, w2, b2)


def _mlp2_body(eps_ref, q0_ref, q1_ref, q2_ref, q3_ref, agg_ref, w1_ref,
               b1_ref, w2_ref, b2_ref, wfc_ref, bfc_ref, out_ref, emb_ref):
  e = 1.0 + eps_ref[0, 0]
  g = jnp.concatenate(
      [q0_ref[...] * e + agg_ref[0], q1_ref[...] * e + agg_ref[1],
       q2_ref[...] * e + agg_ref[2], q3_ref[...] * e + agg_ref[3]],
      axis=1)
  t = jnp.maximum(
      jnp.dot(g, w1_ref[...], preferred_element_type=jnp.float32)
      + b1_ref[...], 0.0)
  h2 = jnp.maximum(
      jnp.dot(t, w2_ref[...], preferred_element_type=jnp.float32)
      + b2_ref[...], 0.0)
  emb_ref[...] = h2
  out_ref[...] = (
      jnp.dot(h2, wfc_ref[...], preferred_element_type=jnp.float32)
      + bfc_ref[...])


def _tc_mlp2(q0, q1, q2, q3, agg, w1, b1, w2, b2, eps, wfc, bfc):
  return pl.pallas_call(
      _mlp2_body,
      grid=(_N // _BLK,),
      in_specs=[
          pl.BlockSpec(memory_space=pltpu.SMEM),
          pl.BlockSpec((_BLK, _W), lambda i: (i, 0)),
          pl.BlockSpec((_BLK, _W), lambda i: (i, 0)),
          pl.BlockSpec((_BLK, _W), lambda i: (i, 0)),
          pl.BlockSpec((_BLK, _W), lambda i: (i, 0)),
          pl.BlockSpec((4, _BLK, _W), lambda i: (0, i, 0)),
          pl.BlockSpec((256, 64), lambda i: (0, 0)),
          pl.BlockSpec((1, 64), lambda i: (0, 0)),
          pl.BlockSpec((64, 64), lambda i: (0, 0)),
          pl.BlockSpec((1, 64), lambda i: (0, 0)),
          pl.BlockSpec((64, 64), lambda i: (0, 0)),
          pl.BlockSpec((1, 64), lambda i: (0, 0)),
      ],
      out_specs=[
          pl.BlockSpec((_BLK, 64), lambda i: (i, 0)),
          pl.BlockSpec((_BLK, 64), lambda i: (i, 0)),
      ],
      out_shape=[
          jax.ShapeDtypeStruct((_N, 64), jnp.float32),
          jax.ShapeDtypeStruct((_N, 64), jnp.float32),
      ],
  )(eps, q0, q1, q2, q3, agg, w1, b1, w2, b2, wfc, bfc)


def kernel(x, edge_index, w1a, b1a, w2a, b2a, eps1, w1b, b1b, w2b, b2b, eps2,
           wfc, bfc):
  ei = edge_index.astype(jnp.int32)
  src2 = _pad_edges(ei[0], 0)
  dst2 = _pad_edges(ei[1], _N)

  xpad = jnp.pad(x, ((0, _NPAD - _N), (0, 0)))
  agg1 = _sc_aggregate(2)(xpad[:, :_W], xpad[:, _W:], src2, dst2)
  q0, q1, q2, q3 = _tc_mlp1(x, agg1, w1a, b1a.reshape(1, -1), w2a,
                            b2a.reshape(1, -1), jnp.reshape(eps1, (1, 1)))
  agg2 = _sc_aggregate(4)(q0, q1, q2, q3, src2, dst2)
  out, emb = _tc_mlp2(q0, q1, q2, q3, agg2, w1b, b1b.reshape(1, -1), w2b,
                      b2b.reshape(1, -1), jnp.reshape(eps2, (1, 1)), wfc,
                      bfc.reshape(1, -1))
  return (out, emb)


# speedup vs baseline: 6.9286x; 1.0173x over previous
import functools

import jax
import jax.numpy as jnp
from jax import lax
from jax.experimental import pallas as pl
from jax.experimental.pallas import tpu as pltpu
from jax.experimental.pallas import tpu_sc as plsc

_N = 10000
_E = 320000
_NS = 16
_CHUNK = 128
_NPAD = 10240
_ZROWS = 128
_RPT = _NPAD // _NS
_NROWS = 2560
_NCH = _NROWS // _NS
_SEG = 32
_W = 64


def _sc_aggregate(nparts: int):
  nphases = nparts // 2
  mesh = plsc.VectorSubcoreMesh(core_axis_name="c", subcore_axis_name="s")

  @functools.partial(
      pl.kernel,
      out_type=jax.ShapeDtypeStruct((nparts, _NPAD, _W), jnp.float32),
      mesh=mesh,
      compiler_params=pltpu.CompilerParams(use_tc_tiling_on_sc=False),
      scratch_types=[
          pltpu.VMEM((_SEG, _CHUNK), jnp.int32),
          pltpu.VMEM((_SEG, _CHUNK), jnp.int32),
          pltpu.VMEM((_CHUNK, _W), jnp.float32),
          pltpu.VMEM((_CHUNK, _W), jnp.float32),
          pltpu.VMEM_SHARED((_NPAD, _W), jnp.float32),
          pltpu.VMEM_SHARED((_NPAD, _W), jnp.float32),
          pltpu.SemaphoreType.DMA,
          pltpu.SemaphoreType.DMA,
          pltpu.SemaphoreType.DMA,
          pltpu.SemaphoreType.DMA,
      ],
  )
  def agg_kernel(*refs):
    qs = refs[:nparts]
    src2, dst2, out = refs[nparts:nparts + 3]
    (src_v, dst_v, rows_a, rows_b, xsp, acc,
     sem_ga, sem_gb, sem_sa, sem_sb) = refs[nparts + 3:]
    c = lax.axis_index("c")
    s = lax.axis_index("s")
    r0 = s * _RPT
    z16 = jnp.zeros((16,), jnp.float32)

    def zrow(r, carry):
      for j in range(_W // 16):
        rows_b[r, pl.ds(j * 16, 16)] = z16
      return carry

    def start_gather(i, rows, sem):
      pltpu.async_copy(xsp.at[src_v.at[i]], rows, sem)

    def wait_gather(rows, sem):
      pltpu.make_async_copy(xsp.at[pl.ds(0, _CHUNK)], rows, sem).wait()

    def start_scatter(i, rows, sem):
      pltpu.async_copy(rows, acc.at[dst_v.at[i]], sem, add=True)

    def wait_scatter(rows, sem):
      pltpu.make_async_copy(rows, acc.at[pl.ds(0, _CHUNK)], sem).wait()

    for phase in range(nphases):
      lax.fori_loop(0, _ZROWS, zrow, 0)
      for jj in range(_RPT // _ZROWS):
        rr = r0 + jj * _ZROWS

        @pl.when(c == 0)
        def _():
          pltpu.sync_copy(qs[2 * phase].at[pl.ds(rr, _ZROWS)], rows_a)

        @pl.when(c == 1)
        def _():
          pltpu.sync_copy(qs[2 * phase + 1].at[pl.ds(rr, _ZROWS)], rows_a)

        pltpu.sync_copy(rows_a, xsp.at[pl.ds(rr, _ZROWS)])
        pltpu.sync_copy(rows_b, acc.at[pl.ds(rr, _ZROWS)])
      plsc.subcore_barrier()

      base = s * _NCH

      def seg_body(g, carry):
        @pl.when(g > 0)
        def _():
          wait_scatter(rows_b, sem_sb)

        row0 = base + g * _SEG
        pltpu.sync_copy(src2.at[pl.ds(row0, _SEG)], src_v)
        pltpu.sync_copy(dst2.at[pl.ds(row0, _SEG)], dst_v)
        start_gather(0, rows_a, sem_ga)

        def pair(p, carry2):
          ia = 2 * p
          ib = 2 * p + 1
          wait_gather(rows_a, sem_ga)
          start_scatter(ia, rows_a, sem_sa)

          @pl.when(p > 0)
          def _():
            wait_scatter(rows_b, sem_sb)

          start_gather(ib, rows_b, sem_gb)
          wait_gather(rows_b, sem_gb)
          start_scatter(ib, rows_b, sem_sb)
          wait_scatter(rows_a, sem_sa)

          @pl.when(ib + 1 < _SEG)
          def _():
            start_gather(ib + 1, rows_a, sem_ga)

          return carry2

        lax.fori_loop(0, _SEG // 2, pair, 0)
        return carry

      lax.fori_loop(0, _NCH // _SEG, seg_body, 0)
      wait_scatter(rows_b, sem_sb)
      plsc.subcore_barrier()

      for jj in range(_RPT // _ZROWS):
        rr = r0 + jj * _ZROWS
        pltpu.sync_copy(acc.at[pl.ds(rr, _ZROWS)], rows_a)

        @pl.when(c == 0)
        def _():
          pltpu.sync_copy(rows_a, out.at[2 * phase, pl.ds(rr, _ZROWS)])

        @pl.when(c == 1)
        def _():
          pltpu.sync_copy(rows_a, out.at[2 * phase + 1, pl.ds(rr, _ZROWS)])

      if phase + 1 < nphases:
        plsc.subcore_barrier()

  return agg_kernel


def _pad_edges(ei):
  pad = _NROWS * _CHUNK - _E
  fills = jnp.array([[0], [_N]], jnp.int32)
  eip = jnp.concatenate(
      [ei, jnp.broadcast_to(fills, (2, pad))], axis=1).reshape(
          2, _NROWS, _CHUNK)
  return eip[0], eip[1]


_BLK = 2000


def _mlp1_body(eps_ref, x_ref, agg_ref, w1_ref, b1_ref, w2_ref, b2_ref,
               q0_ref, q1_ref, q2_ref, q3_ref):
  g = (x_ref[...] * (1.0 + eps_ref[0, 0])
       + jnp.concatenate([agg_ref[0], agg_ref[1]], axis=1))
  t = jnp.maximum(
      jnp.dot(g, w1_ref[...], preferred_element_type=jnp.float32)
      + b1_ref[...], 0.0)
  h = jnp.maximum(
      jnp.dot(t, w2_ref[...], preferred_element_type=jnp.float32)
      + b2_ref[...], 0.0)
  q0_ref[...] = h[:, 0:64]
  q1_ref[...] = h[:, 64:128]
  q2_ref[...] = h[:, 128:192]
  q3_ref[...] = h[:, 192:256]


def _tc_mlp1(x, agg, w1, b1, w2, b2, eps):
  return pl.pallas_call(
      _mlp1_body,
      grid=(_N // _BLK,),
      in_specs=[
          pl.BlockSpec(memory_space=pltpu.SMEM),
          pl.BlockSpec((_BLK, 128), lambda i: (i, 0)),
          pl.BlockSpec((2, _BLK, _W), lambda i: (0, i, 0)),
          pl.BlockSpec((128, 256), lambda i: (0, 0)),
          pl.BlockSpec((1, 256), lambda i: (0, 0)),
          pl.BlockSpec((256, 256), lambda i: (0, 0)),
          pl.BlockSpec((1, 256), lambda i: (0, 0)),
      ],
      out_specs=[
          pl.BlockSpec((_BLK, _W), lambda i: (i, 0)),
          pl.BlockSpec((_BLK, _W), lambda i: (i, 0)),
          pl.BlockSpec((_BLK, _W), lambda i: (i, 0)),
          pl.BlockSpec((_BLK, _W), lambda i: (i, 0)),
      ],
      out_shape=[
          jax.ShapeDtypeStruct((_NPAD, _W), jnp.float32),
          jax.ShapeDtypeStruct((_NPAD, _W), jnp.float32),
          jax.ShapeDtypeStruct((_NPAD, _W), jnp.float32),
          jax.ShapeDtypeStruct((_NPAD, _W), jnp.float32),
      ],
  )(eps, x, agg, w1, b1, w2, b2)


def _mlp2_body(eps_ref, q0_ref, q1_ref, q2_ref, q3_ref, agg_ref, w1_ref,
               b1_ref, w2_ref, b2_ref, wfc_ref, bfc_ref, out_ref, emb_ref):
  e = 1.0 + eps_ref[0, 0]
  g = jnp.concatenate(
      [q0_ref[...] * e + agg_ref[0], q1_ref[...] * e + agg_ref[1],
       q2_ref[...] * e + agg_ref[2], q3_ref[...] * e + agg_ref[3]],
      axis=1)
  t = jnp.maximum(
      jnp.dot(g, w1_ref[...], preferred_element_type=jnp.float32)
      + b1_ref[...], 0.0)
  h2 = jnp.maximum(
      jnp.dot(t, w2_ref[...], preferred_element_type=jnp.float32)
      + b2_ref[...], 0.0)
  emb_ref[...] = h2
  out_ref[...] = (
      jnp.dot(h2, wfc_ref[...], preferred_element_type=jnp.float32)
      + bfc_ref[...])


def _tc_mlp2(q0, q1, q2, q3, agg, w1, b1, w2, b2, eps, wfc, bfc):
  return pl.pallas_call(
      _mlp2_body,
      grid=(_N // _BLK,),
      in_specs=[
          pl.BlockSpec(memory_space=pltpu.SMEM),
          pl.BlockSpec((_BLK, _W), lambda i: (i, 0)),
          pl.BlockSpec((_BLK, _W), lambda i: (i, 0)),
          pl.BlockSpec((_BLK, _W), lambda i: (i, 0)),
          pl.BlockSpec((_BLK, _W), lambda i: (i, 0)),
          pl.BlockSpec((4, _BLK, _W), lambda i: (0, i, 0)),
          pl.BlockSpec((256, 64), lambda i: (0, 0)),
          pl.BlockSpec((1, 64), lambda i: (0, 0)),
          pl.BlockSpec((64, 64), lambda i: (0, 0)),
          pl.BlockSpec((1, 64), lambda i: (0, 0)),
          pl.BlockSpec((64, 64), lambda i: (0, 0)),
          pl.BlockSpec((1, 64), lambda i: (0, 0)),
      ],
      out_specs=[
          pl.BlockSpec((_BLK, 64), lambda i: (i, 0)),
          pl.BlockSpec((_BLK, 64), lambda i: (i, 0)),
      ],
      out_shape=[
          jax.ShapeDtypeStruct((_N, 64), jnp.float32),
          jax.ShapeDtypeStruct((_N, 64), jnp.float32),
      ],
  )(eps, q0, q1, q2, q3, agg, w1, b1, w2, b2, wfc, bfc)


def kernel(x, edge_index, w1a, b1a, w2a, b2a, eps1, w1b, b1b, w2b, b2b, eps2,
           wfc, bfc):
  src2, dst2 = _pad_edges(edge_index.astype(jnp.int32))

  xpad = jnp.pad(x, ((0, _NPAD - _N), (0, 0)))
  agg1 = _sc_aggregate(2)(xpad[:, :_W], xpad[:, _W:], src2, dst2)
  q0, q1, q2, q3 = _tc_mlp1(x, agg1, w1a, b1a.reshape(1, -1), w2a,
                            b2a.reshape(1, -1), jnp.reshape(eps1, (1, 1)))
  agg2 = _sc_aggregate(4)(q0, q1, q2, q3, src2, dst2)
  out, emb = _tc_mlp2(q0, q1, q2, q3, agg2, w1b, b1b.reshape(1, -1), w2b,
                      b2b.reshape(1, -1), jnp.reshape(eps2, (1, 1)), wfc,
                      bfc.reshape(1, -1))
  return (out, emb)


# speedup vs baseline: 7.1770x; 1.0358x over previous
import functools

import jax
import jax.numpy as jnp
from jax import lax
from jax.experimental import pallas as pl
from jax.experimental.pallas import tpu as pltpu
from jax.experimental.pallas import tpu_sc as plsc

_N = 10000
_E = 320000
_NS = 16
_CHUNK = 128
_NPAD = 10240
_ZROWS = 128
_RPT = _NPAD // _NS
_NROWS = 2560
_NCH = _NROWS // _NS
_SEG = 32
_W = 64


def _sc_aggregate(nparts: int):
  nphases = nparts // 2
  mesh = plsc.VectorSubcoreMesh(core_axis_name="c", subcore_axis_name="s")

  @functools.partial(
      pl.kernel,
      out_type=jax.ShapeDtypeStruct((nparts, _NPAD, _W), jnp.float32),
      mesh=mesh,
      compiler_params=pltpu.CompilerParams(use_tc_tiling_on_sc=False),
      scratch_types=[
          pltpu.VMEM((2, _SEG, _CHUNK), jnp.int32),
          pltpu.VMEM((2, _SEG, _CHUNK), jnp.int32),
          pltpu.VMEM((_CHUNK, _W), jnp.float32),
          pltpu.VMEM((_CHUNK, _W), jnp.float32),
          pltpu.VMEM_SHARED((_NPAD, _W), jnp.float32),
          pltpu.VMEM_SHARED((_NPAD, _W), jnp.float32),
          pltpu.SemaphoreType.DMA,
          pltpu.SemaphoreType.DMA,
          pltpu.SemaphoreType.DMA,
          pltpu.SemaphoreType.DMA,
          pltpu.SemaphoreType.DMA,
      ],
  )
  def agg_kernel(*refs):
    qs = refs[:nparts]
    src2, dst2, out = refs[nparts:nparts + 3]
    (src_v, dst_v, rows_a, rows_b, xsp, acc,
     sem_ga, sem_gb, sem_sa, sem_sb, sem_i) = refs[nparts + 3:]
    c = lax.axis_index("c")
    s = lax.axis_index("s")
    r0 = s * _RPT
    z16 = jnp.zeros((16,), jnp.float32)

    def zrow(r, carry):
      for j in range(_W // 16):
        rows_b[r, pl.ds(j * 16, 16)] = z16
      return carry

    def start_idx(g):
      row0 = s * _NCH + g * _SEG
      par = g & 1
      pltpu.async_copy(src2.at[pl.ds(row0, _SEG)], src_v.at[par], sem_i)
      pltpu.async_copy(dst2.at[pl.ds(row0, _SEG)], dst_v.at[par], sem_i)

    def wait_idx():
      for _ in range(2):
        pltpu.make_async_copy(
            src2.at[pl.ds(0, _SEG)], src_v.at[0], sem_i).wait()

    def wait_gather(rows, sem):
      pltpu.make_async_copy(xsp.at[pl.ds(0, _CHUNK)], rows, sem).wait()

    def wait_scatter(rows, sem):
      pltpu.make_async_copy(rows, acc.at[pl.ds(0, _CHUNK)], sem).wait()

    for phase in range(nphases):
      lax.fori_loop(0, _ZROWS, zrow, 0)
      for jj in range(_RPT // _ZROWS):
        rr = r0 + jj * _ZROWS

        @pl.when(c == 0)
        def _():
          pltpu.sync_copy(qs[2 * phase].at[pl.ds(rr, _ZROWS)], rows_a)

        @pl.when(c == 1)
        def _():
          pltpu.sync_copy(qs[2 * phase + 1].at[pl.ds(rr, _ZROWS)], rows_a)

        pltpu.sync_copy(rows_a, xsp.at[pl.ds(rr, _ZROWS)])
        pltpu.sync_copy(rows_b, acc.at[pl.ds(rr, _ZROWS)])
      plsc.subcore_barrier()

      def seg_body(g, carry):
        par = g & 1

        @pl.when(g > 0)
        def _():
          wait_scatter(rows_b, sem_sb)

        wait_idx()

        @pl.when(g + 1 < _NCH // _SEG)
        def _():
          start_idx(g + 1)

        def start_gather(i, rows, sem):
          pltpu.async_copy(xsp.at[src_v.at[par, i]], rows, sem)

        def start_scatter(i, rows, sem):
          pltpu.async_copy(rows, acc.at[dst_v.at[par, i]], sem, add=True)

        start_gather(0, rows_a, sem_ga)

        def pair(p, carry2):
          ia = 2 * p
          ib = 2 * p + 1
          wait_gather(rows_a, sem_ga)
          start_scatter(ia, rows_a, sem_sa)

          @pl.when(p > 0)
          def _():
            wait_scatter(rows_b, sem_sb)

          start_gather(ib, rows_b, sem_gb)
          wait_gather(rows_b, sem_gb)
          start_scatter(ib, rows_b, sem_sb)
          wait_scatter(rows_a, sem_sa)

          @pl.when(ib + 1 < _SEG)
          def _():
            start_gather(ib + 1, rows_a, sem_ga)

          return carry2

        lax.fori_loop(0, _SEG // 2, pair, 0)
        return carry

      start_idx(0)
      lax.fori_loop(0, _NCH // _SEG, seg_body, 0)
      wait_scatter(rows_b, sem_sb)
      plsc.subcore_barrier()

      for jj in range(_RPT // _ZROWS):
        rr = r0 + jj * _ZROWS
        pltpu.sync_copy(acc.at[pl.ds(rr, _ZROWS)], rows_a)

        @pl.when(c == 0)
        def _():
          pltpu.sync_copy(rows_a, out.at[2 * phase, pl.ds(rr, _ZROWS)])

        @pl.when(c == 1)
        def _():
          pltpu.sync_copy(rows_a, out.at[2 * phase + 1, pl.ds(rr, _ZROWS)])

      if phase + 1 < nphases:
        plsc.subcore_barrier()

  return agg_kernel


def _pad_edges(ei):
  pad = _NROWS * _CHUNK - _E
  fills = jnp.array([[0], [_N]], jnp.int32)
  eip = jnp.concatenate(
      [ei, jnp.broadcast_to(fills, (2, pad))], axis=1).reshape(
          2, _NROWS, _CHUNK)
  return eip[0], eip[1]


_BLK = 2000


def _mlp1_body(eps_ref, x_ref, agg_ref, w1_ref, b1_ref, w2_ref, b2_ref,
               q0_ref, q1_ref, q2_ref, q3_ref):
  g = (x_ref[...] * (1.0 + eps_ref[0, 0])
       + jnp.concatenate([agg_ref[0], agg_ref[1]], axis=1))
  t = jnp.maximum(
      jnp.dot(g, w1_ref[...], preferred_element_type=jnp.float32)
      + b1_ref[...], 0.0)
  h = jnp.maximum(
      jnp.dot(t, w2_ref[...], preferred_element_type=jnp.float32)
      + b2_ref[...], 0.0)
  q0_ref[...] = h[:, 0:64]
  q1_ref[...] = h[:, 64:128]
  q2_ref[...] = h[:, 128:192]
  q3_ref[...] = h[:, 192:256]


def _tc_mlp1(x, agg, w1, b1, w2, b2, eps):
  return pl.pallas_call(
      _mlp1_body,
      grid=(_N // _BLK,),
      in_specs=[
          pl.BlockSpec(memory_space=pltpu.SMEM),
          pl.BlockSpec((_BLK, 128), lambda i: (i, 0)),
          pl.BlockSpec((2, _BLK, _W), lambda i: (0, i, 0)),
          pl.BlockSpec((128, 256), lambda i: (0, 0)),
          pl.BlockSpec((1, 256), lambda i: (0, 0)),
          pl.BlockSpec((256, 256), lambda i: (0, 0)),
          pl.BlockSpec((1, 256), lambda i: (0, 0)),
      ],
      out_specs=[
          pl.BlockSpec((_BLK, _W), lambda i: (i, 0)),
          pl.BlockSpec((_BLK, _W), lambda i: (i, 0)),
          pl.BlockSpec((_BLK, _W), lambda i: (i, 0)),
          pl.BlockSpec((_BLK, _W), lambda i: (i, 0)),
      ],
      out_shape=[
          jax.ShapeDtypeStruct((_NPAD, _W), jnp.float32),
          jax.ShapeDtypeStruct((_NPAD, _W), jnp.float32),
          jax.ShapeDtypeStruct((_NPAD, _W), jnp.float32),
          jax.ShapeDtypeStruct((_NPAD, _W), jnp.float32),
      ],
  )(eps, x, agg, w1, b1, w2, b2)


def _mlp2_body(eps_ref, q0_ref, q1_ref, q2_ref, q3_ref, agg_ref, w1_ref,
               b1_ref, w2_ref, b2_ref, wfc_ref, bfc_ref, out_ref, emb_ref):
  e = 1.0 + eps_ref[0, 0]
  g = jnp.concatenate(
      [q0_ref[...] * e + agg_ref[0], q1_ref[...] * e + agg_ref[1],
       q2_ref[...] * e + agg_ref[2], q3_ref[...] * e + agg_ref[3]],
      axis=1)
  t = jnp.maximum(
      jnp.dot(g, w1_ref[...], preferred_element_type=jnp.float32)
      + b1_ref[...], 0.0)
  h2 = jnp.maximum(
      jnp.dot(t, w2_ref[...], preferred_element_type=jnp.float32)
      + b2_ref[...], 0.0)
  emb_ref[...] = h2
  out_ref[...] = (
      jnp.dot(h2, wfc_ref[...], preferred_element_type=jnp.float32)
      + bfc_ref[...])


def _tc_mlp2(q0, q1, q2, q3, agg, w1, b1, w2, b2, eps, wfc, bfc):
  return pl.pallas_call(
      _mlp2_body,
      grid=(_N // _BLK,),
      in_specs=[
          pl.BlockSpec(memory_space=pltpu.SMEM),
          pl.BlockSpec((_BLK, _W), lambda i: (i, 0)),
          pl.BlockSpec((_BLK, _W), lambda i: (i, 0)),
          pl.BlockSpec((_BLK, _W), lambda i: (i, 0)),
          pl.BlockSpec((_BLK, _W), lambda i: (i, 0)),
          pl.BlockSpec((4, _BLK, _W), lambda i: (0, i, 0)),
          pl.BlockSpec((256, 64), lambda i: (0, 0)),
          pl.BlockSpec((1, 64), lambda i: (0, 0)),
          pl.BlockSpec((64, 64), lambda i: (0, 0)),
          pl.BlockSpec((1, 64), lambda i: (0, 0)),
          pl.BlockSpec((64, 64), lambda i: (0, 0)),
          pl.BlockSpec((1, 64), lambda i: (0, 0)),
      ],
      out_specs=[
          pl.BlockSpec((_BLK, 64), lambda i: (i, 0)),
          pl.BlockSpec((_BLK, 64), lambda i: (i, 0)),
      ],
      out_shape=[
          jax.ShapeDtypeStruct((_N, 64), jnp.float32),
          jax.ShapeDtypeStruct((_N, 64), jnp.float32),
      ],
  )(eps, q0, q1, q2, q3, agg, w1, b1, w2, b2, wfc, bfc)


def kernel(x, edge_index, w1a, b1a, w2a, b2a, eps1, w1b, b1b, w2b, b2b, eps2,
           wfc, bfc):
  src2, dst2 = _pad_edges(edge_index.astype(jnp.int32))

  xpad = jnp.pad(x, ((0, _NPAD - _N), (0, 0)))
  agg1 = _sc_aggregate(2)(xpad[:, :_W], xpad[:, _W:], src2, dst2)
  q0, q1, q2, q3 = _tc_mlp1(x, agg1, w1a, b1a.reshape(1, -1), w2a,
                            b2a.reshape(1, -1), jnp.reshape(eps1, (1, 1)))
  agg2 = _sc_aggregate(4)(q0, q1, q2, q3, src2, dst2)
  out, emb = _tc_mlp2(q0, q1, q2, q3, agg2, w1b, b1b.reshape(1, -1), w2b,
                      b2b.reshape(1, -1), jnp.reshape(eps2, (1, 1)), wfc,
                      bfc.reshape(1, -1))
  return (out, emb)


# speedup vs baseline: 7.3386x; 1.0225x over previous
import functools

import jax
import jax.numpy as jnp
from jax import lax
from jax.experimental import pallas as pl
from jax.experimental.pallas import tpu as pltpu
from jax.experimental.pallas import tpu_sc as plsc

_N = 10000
_E = 320000
_NS = 16
_CHUNK = 128
_NPAD = 10240
_ZROWS = 128
_RPT = _NPAD // _NS
_NROWS = 2560
_NCH = _NROWS // _NS
_SEG = 32
_W = 64


def _sc_aggregate(nparts: int):
  nphases = nparts // 2
  mesh = plsc.VectorSubcoreMesh(core_axis_name="c", subcore_axis_name="s")

  @functools.partial(
      pl.kernel,
      out_type=jax.ShapeDtypeStruct((nparts, _NPAD, _W), jnp.float32),
      mesh=mesh,
      compiler_params=pltpu.CompilerParams(use_tc_tiling_on_sc=False),
      scratch_types=[
          pltpu.VMEM((2, _SEG, _CHUNK), jnp.int32),
          pltpu.VMEM((2, _SEG, _CHUNK), jnp.int32),
          pltpu.VMEM((_CHUNK, _W), jnp.float32),
          pltpu.VMEM((_CHUNK, _W), jnp.float32),
          pltpu.VMEM((_ZROWS, _W), jnp.float32),
          pltpu.VMEM_SHARED((_NPAD, _W), jnp.float32),
          pltpu.VMEM_SHARED((_NPAD, _W), jnp.float32),
          pltpu.SemaphoreType.DMA,
          pltpu.SemaphoreType.DMA,
          pltpu.SemaphoreType.DMA,
          pltpu.SemaphoreType.DMA,
          pltpu.SemaphoreType.DMA,
      ],
  )
  def agg_kernel(*refs):
    qs = refs[:nparts]
    src2, dst2, out = refs[nparts:nparts + 3]
    (src_v, dst_v, rows_a, rows_b, zbuf, xsp, acc,
     sem_ga, sem_gb, sem_sa, sem_sb, sem_i) = refs[nparts + 3:]
    c = lax.axis_index("c")
    s = lax.axis_index("s")
    r0 = s * _RPT
    z16 = jnp.zeros((16,), jnp.float32)

    def zrow(r, carry):
      for j in range(_W // 16):
        zbuf[r, pl.ds(j * 16, 16)] = z16
      return carry

    lax.fori_loop(0, _ZROWS, zrow, 0)

    def start_idx(g):
      row0 = s * _NCH + g * _SEG
      par = g & 1
      pltpu.async_copy(src2.at[pl.ds(row0, _SEG)], src_v.at[par], sem_i)
      pltpu.async_copy(dst2.at[pl.ds(row0, _SEG)], dst_v.at[par], sem_i)

    def wait_idx():
      for _ in range(2):
        pltpu.make_async_copy(
            src2.at[pl.ds(0, _SEG)], src_v.at[0], sem_i).wait()

    def wait_gather(rows, sem):
      pltpu.make_async_copy(xsp.at[pl.ds(0, _CHUNK)], rows, sem).wait()

    def wait_scatter(rows, sem):
      pltpu.make_async_copy(rows, acc.at[pl.ds(0, _CHUNK)], sem).wait()

    for phase in range(nphases):
      for jj in range(_RPT // _ZROWS):
        rr = r0 + jj * _ZROWS

        @pl.when(c == 0)
        def _():
          pltpu.sync_copy(qs[2 * phase].at[pl.ds(rr, _ZROWS)], rows_a)

        @pl.when(c == 1)
        def _():
          pltpu.sync_copy(qs[2 * phase + 1].at[pl.ds(rr, _ZROWS)], rows_a)

        pltpu.sync_copy(rows_a, xsp.at[pl.ds(rr, _ZROWS)])
        pltpu.sync_copy(zbuf, acc.at[pl.ds(rr, _ZROWS)])
      plsc.subcore_barrier()

      def seg_body(g, carry):
        par = g & 1

        @pl.when(g > 0)
        def _():
          wait_scatter(rows_b, sem_sb)

        wait_idx()

        @pl.when(g + 1 < _NCH // _SEG)
        def _():
          start_idx(g + 1)

        def start_gather(i, rows, sem):
          pltpu.async_copy(xsp.at[src_v.at[par, i]], rows, sem)

        def start_scatter(i, rows, sem):
          pltpu.async_copy(rows, acc.at[dst_v.at[par, i]], sem, add=True)

        start_gather(0, rows_a, sem_ga)

        def pair(p, carry2):
          ia = 2 * p
          ib = 2 * p + 1
          wait_gather(rows_a, sem_ga)
          start_scatter(ia, rows_a, sem_sa)

          @pl.when(p > 0)
          def _():
            wait_scatter(rows_b, sem_sb)

          start_gather(ib, rows_b, sem_gb)
          wait_gather(rows_b, sem_gb)
          start_scatter(ib, rows_b, sem_sb)
          wait_scatter(rows_a, sem_sa)

          @pl.when(ib + 1 < _SEG)
          def _():
            start_gather(ib + 1, rows_a, sem_ga)

          return carry2

        lax.fori_loop(0, _SEG // 2, pair, 0)
        return carry

      start_idx(0)
      lax.fori_loop(0, _NCH // _SEG, seg_body, 0)
      wait_scatter(rows_b, sem_sb)
      plsc.subcore_barrier()

      for jj in range(_RPT // _ZROWS):
        rr = r0 + jj * _ZROWS
        pltpu.sync_copy(acc.at[pl.ds(rr, _ZROWS)], rows_a)

        @pl.when(c == 0)
        def _():
          pltpu.sync_copy(rows_a, out.at[2 * phase, pl.ds(rr, _ZROWS)])

        @pl.when(c == 1)
        def _():
          pltpu.sync_copy(rows_a, out.at[2 * phase + 1, pl.ds(rr, _ZROWS)])

      if phase + 1 < nphases:
        plsc.subcore_barrier()

  return agg_kernel


def _pad_edges(ei):
  pad = _NROWS * _CHUNK - _E
  fills = jnp.array([[0], [_N]], jnp.int32)
  eip = jnp.concatenate(
      [ei, jnp.broadcast_to(fills, (2, pad))], axis=1).reshape(
          2, _NROWS, _CHUNK)
  return eip[0], eip[1]


_BLK = 2000


def _mlp1_body(eps_ref, x_ref, agg_ref, w1_ref, b1_ref, w2_ref, b2_ref,
               q0_ref, q1_ref, q2_ref, q3_ref):
  g = (x_ref[...] * (1.0 + eps_ref[0, 0])
       + jnp.concatenate([agg_ref[0], agg_ref[1]], axis=1))
  t = jnp.maximum(
      jnp.dot(g, w1_ref[...], preferred_element_type=jnp.float32)
      + b1_ref[...], 0.0)
  h = jnp.maximum(
      jnp.dot(t, w2_ref[...], preferred_element_type=jnp.float32)
      + b2_ref[...], 0.0)
  q0_ref[...] = h[:, 0:64]
  q1_ref[...] = h[:, 64:128]
  q2_ref[...] = h[:, 128:192]
  q3_ref[...] = h[:, 192:256]


def _tc_mlp1(x, agg, w1, b1, w2, b2, eps):
  return pl.pallas_call(
      _mlp1_body,
      grid=(_N // _BLK,),
      in_specs=[
          pl.BlockSpec(memory_space=pltpu.SMEM),
          pl.BlockSpec((_BLK, 128), lambda i: (i, 0)),
          pl.BlockSpec((2, _BLK, _W), lambda i: (0, i, 0)),
          pl.BlockSpec((128, 256), lambda i: (0, 0)),
          pl.BlockSpec((1, 256), lambda i: (0, 0)),
          pl.BlockSpec((256, 256), lambda i: (0, 0)),
          pl.BlockSpec((1, 256), lambda i: (0, 0)),
      ],
      out_specs=[
          pl.BlockSpec((_BLK, _W), lambda i: (i, 0)),
          pl.BlockSpec((_BLK, _W), lambda i: (i, 0)),
          pl.BlockSpec((_BLK, _W), lambda i: (i, 0)),
          pl.BlockSpec((_BLK, _W), lambda i: (i, 0)),
      ],
      out_shape=[
          jax.ShapeDtypeStruct((_NPAD, _W), jnp.float32),
          jax.ShapeDtypeStruct((_NPAD, _W), jnp.float32),
          jax.ShapeDtypeStruct((_NPAD, _W), jnp.float32),
          jax.ShapeDtypeStruct((_NPAD, _W), jnp.float32),
      ],
  )(eps, x, agg, w1, b1, w2, b2)


def _mlp2_body(eps_ref, q0_ref, q1_ref, q2_ref, q3_ref, agg_ref, w1_ref,
               b1_ref, w2_ref, b2_ref, wfc_ref, bfc_ref, out_ref, emb_ref):
  e = 1.0 + eps_ref[0, 0]
  g = jnp.concatenate(
      [q0_ref[...] * e + agg_ref[0], q1_ref[...] * e + agg_ref[1],
       q2_ref[...] * e + agg_ref[2], q3_ref[...] * e + agg_ref[3]],
      axis=1)
  t = jnp.maximum(
      jnp.dot(g, w1_ref[...], preferred_element_type=jnp.float32)
      + b1_ref[...], 0.0)
  h2 = jnp.maximum(
      jnp.dot(t, w2_ref[...], preferred_element_type=jnp.float32)
      + b2_ref[...], 0.0)
  emb_ref[...] = h2
  out_ref[...] = (
      jnp.dot(h2, wfc_ref[...], preferred_element_type=jnp.float32)
      + bfc_ref[...])


def _tc_mlp2(q0, q1, q2, q3, agg, w1, b1, w2, b2, eps, wfc, bfc):
  return pl.pallas_call(
      _mlp2_body,
      grid=(_N // _BLK,),
      in_specs=[
          pl.BlockSpec(memory_space=pltpu.SMEM),
          pl.BlockSpec((_BLK, _W), lambda i: (i, 0)),
          pl.BlockSpec((_BLK, _W), lambda i: (i, 0)),
          pl.BlockSpec((_BLK, _W), lambda i: (i, 0)),
          pl.BlockSpec((_BLK, _W), lambda i: (i, 0)),
          pl.BlockSpec((4, _BLK, _W), lambda i: (0, i, 0)),
          pl.BlockSpec((256, 64), lambda i: (0, 0)),
          pl.BlockSpec((1, 64), lambda i: (0, 0)),
          pl.BlockSpec((64, 64), lambda i: (0, 0)),
          pl.BlockSpec((1, 64), lambda i: (0, 0)),
          pl.BlockSpec((64, 64), lambda i: (0, 0)),
          pl.BlockSpec((1, 64), lambda i: (0, 0)),
      ],
      out_specs=[
          pl.BlockSpec((_BLK, 64), lambda i: (i, 0)),
          pl.BlockSpec((_BLK, 64), lambda i: (i, 0)),
      ],
      out_shape=[
          jax.ShapeDtypeStruct((_N, 64), jnp.float32),
          jax.ShapeDtypeStruct((_N, 64), jnp.float32),
      ],
  )(eps, q0, q1, q2, q3, agg, w1, b1, w2, b2, wfc, bfc)


def kernel(x, edge_index, w1a, b1a, w2a, b2a, eps1, w1b, b1b, w2b, b2b, eps2,
           wfc, bfc):
  src2, dst2 = _pad_edges(edge_index.astype(jnp.int32))

  xpad = jnp.pad(x, ((0, _NPAD - _N), (0, 0)))
  agg1 = _sc_aggregate(2)(xpad[:, :_W], xpad[:, _W:], src2, dst2)
  q0, q1, q2, q3 = _tc_mlp1(x, agg1, w1a, b1a.reshape(1, -1), w2a,
                            b2a.reshape(1, -1), jnp.reshape(eps1, (1, 1)))
  agg2 = _sc_aggregate(4)(q0, q1, q2, q3, src2, dst2)
  out, emb = _tc_mlp2(q0, q1, q2, q3, agg2, w1b, b1b.reshape(1, -1), w2b,
                      b2b.reshape(1, -1), jnp.reshape(eps2, (1, 1)), wfc,
                      bfc.reshape(1, -1))
  return (out, emb)
